# baseline (speedup 1.0000x reference)
import functools

import jax
import jax.numpy as jnp
from jax import lax
from jax.experimental import pallas as pl
from jax.experimental.pallas import tpu as pltpu
from jax.experimental.pallas import tpu_sc as plsc

_NC = 2
_NS = 16
_NW = _NC * _NS
_PAD = 1024


_SPMEM_WORDS = 2097151


def _buf_sizes(acc_rows, Vw, C, nnz):
    ZR = 1
    for d in range(1, Vw + 1):
        if Vw % d == 0 and d * C <= 1024:
            ZR = d
    acc = acc_rows * C
    tile_words = (_SPMEM_WORDS - acc - 49152) // 16
    avail = tile_words - ZR * C - 512
    G = min(128, (avail // (2 * C + 31)) // 16 * 16)
    if G < 16:
        raise ValueError("no TileSpmem budget for acc_rows=%d C=%d" % (acc_rows, C))
    SCN = 2 if nnz <= 2 * _NW * G else 4
    return G, ZR, SCN


@functools.lru_cache(maxsize=None)
def _make_spmm(V, C, nnzp):
    V4 = V // 4
    VH = V // 2
    Vw = V // (2 * _NW)
    G, ZR, SCN = _buf_sizes(V4, Vw, C, nnzp - _PAD)
    NSUB = G // 16
    q = Vw // ZR
    EB = SCN * G
    NCG = C // 16

    mesh = plsc.VectorSubcoreMesh(core_axis_name="c", subcore_axis_name="s")

    def body(x_hbm, cols_hbm, rows_hbm, vals_hbm, meta_hbm, y_hbm,
             acc, g0, g1, ecols, erows, evals, cb0, cb1, vs, lr0, lr1,
             zbuf, metab, sem0, sem1, ssc0, ssc1):
        cid = lax.axis_index("c")
        sid = lax.axis_index("s")
        w = cid * _NS + sid
        lane = lax.iota(jnp.int32, 16)

        pltpu.sync_copy(meta_hbm, metab)

        def ext(base):
            seg0 = metab[pl.ds(base, 16)]
            seg1 = metab[pl.ds(base + 16, 16)]
            seg = jnp.where(jnp.full((16,), cid, jnp.int32) == 0, seg0, seg1)
            return jnp.sum(jnp.where(lane == sid, seg, 0))

        z16 = jnp.zeros((16,), jnp.float32)

        def zrow(r, _):
            def zcol(j, _):
                zbuf[r, pl.ds(j * 16, 16)] = z16
                return 0
            return lax.fori_loop(0, NCG, zcol, 0)
        lax.fori_loop(0, ZR, zrow, 0)

        rb = [lane + s * 16 for s in range(NSUB)]
        own_lo = sid * Vw
        gbuf = [g0, g1]
        cbuf = [cb0, cb1]
        lrb = [lr0, lr1]
        sems = [sem0, sem1]
        sscs = [ssc0, ssc1]

        for h in range(2):
            a0 = ext(h * 96)
            b0 = ext(h * 96 + 32)
            b1 = ext(h * 96 + 64)
            nch = (b1 - a0 + (G - 1)) // G
            nsup = (nch + (SCN - 1)) // SCN
            rbase = h * VH + cid * V4

            def load_block(sup):
                off = pl.multiple_of(a0 + sup * EB, 8)
                pltpu.sync_copy(cols_hbm.at[pl.ds(off, EB)], ecols)
                pltpu.sync_copy(rows_hbm.at[pl.ds(off, EB)], erows)
                pltpu.sync_copy(vals_hbm.at[pl.ds(off, EB)], evals)

            def issue(m, p):
                for s in range(NSUB):
                    cbuf[p][pl.ds(s * 16, 16)] = ecols[pl.ds(m * G + s * 16, 16)]
                pltpu.async_copy(x_hbm.at[cbuf[p]], gbuf[p], sems[p])

            def zcp(i, _):
                pltpu.sync_copy(zbuf, acc.at[pl.ds(own_lo + i * ZR, ZR)])
                return 0
            lax.fori_loop(0, q, zcp, 0)

            @pl.when(nsup > 0)
            def _():
                load_block(0)
                issue(0, 0)

            def sup_body(sup, _):
                for m in range(SCN):
                    p = m % 2
                    pltpu.make_async_copy(
                        x_hbm.at[cbuf[p]], gbuf[p], sems[p]).wait()

                    base = a0 + (sup * SCN + m) * G
                    for s in range(NSUB):
                        ei = base + s * 16 + lane
                        valid = (ei >= b0) & (ei < b1)
                        vv = jnp.where(
                            valid, evals[pl.ds(m * G + s * 16, 16)], 0.0)
                        vs[pl.ds(s * 16, 16)] = vv
                        rr = erows[pl.ds(m * G + s * 16, 16)] - rbase
                        rr = jnp.clip(rr, own_lo, own_lo + (Vw - 1))
                        lrb[p][pl.ds(s * 16, 16)] = rr

                    def wait_sc(pp):
                        pltpu.make_async_copy(
                            gbuf[pp], acc.at[lrb[pp]], sscs[pp]).wait()

                    if m == 0:
                        @pl.when(sup > 0)
                        def _():
                            wait_sc(1 - p)
                        issue(m + 1, 1 - p)
                    elif m < SCN - 1:
                        wait_sc(1 - p)
                        issue(m + 1, 1 - p)
                    else:
                        @pl.when(sup + 1 < nsup)
                        def _():
                            wait_sc(1 - p)
                            load_block(sup + 1)
                            issue(0, 1 - p)

                    gp = gbuf[p]

                    @plsc.parallel_loop(0, G, unroll=4)
                    def _(j):
                        vsj = plsc.load_gather(
                            vs, [jnp.full((16,), j, jnp.int32)])
                        for cg in range(NCG):
                            gp[j, pl.ds(cg * 16, 16)] = (
                                gp[j, pl.ds(cg * 16, 16)] * vsj)

                    pltpu.async_copy(
                        gbuf[p], acc.at[lrb[p]], sscs[p], add=True)
                return 0
            lax.fori_loop(0, nsup, sup_body, 0)

            @pl.when(nsup > 0)
            def _():
                pltpu.make_async_copy(gbuf[0], acc.at[lrb[0]], sscs[0]).wait()
                pltpu.make_async_copy(gbuf[1], acc.at[lrb[1]], sscs[1]).wait()

            pltpu.sync_copy(acc.at[pl.ds(own_lo, Vw)],
                            y_hbm.at[pl.ds(h * VH + w * Vw, Vw)])

    return pl.kernel(
        body,
        out_type=jax.ShapeDtypeStruct((V, C), jnp.float32),
        mesh=mesh,
        scratch_types=[
            pltpu.VMEM_SHARED((V4, C), jnp.float32),
            pltpu.VMEM((G, C), jnp.float32),
            pltpu.VMEM((G, C), jnp.float32),
            pltpu.VMEM((EB,), jnp.int32),
            pltpu.VMEM((EB,), jnp.int32),
            pltpu.VMEM((EB,), jnp.float32),
            pltpu.VMEM((G,), jnp.int32),
            pltpu.VMEM((G,), jnp.int32),
            pltpu.VMEM((G,), jnp.float32),
            pltpu.VMEM((G,), jnp.int32),
            pltpu.VMEM((G,), jnp.int32),
            pltpu.VMEM((ZR, C), jnp.float32),
            pltpu.VMEM((192,), jnp.int32),
            pltpu.SemaphoreType.DMA,
            pltpu.SemaphoreType.DMA,
            pltpu.SemaphoreType.DMA,
            pltpu.SemaphoreType.DMA,
        ],
        compiler_params=pltpu.CompilerParams(use_tc_tiling_on_sc=False, needs_layout_passes=False),
    )


@functools.lru_cache(maxsize=None)
def _make_mm3(M, K, N, act, Mb):
    def body(x0, s1, s2, wr, br, o):
        for b in range(2):
            cs = pl.ds(b * K, K)
            z = jnp.dot(x0[:, cs], wr[0:K, :],
                        preferred_element_type=jnp.float32)
            z += jnp.dot(s1[:, cs], wr[K:2 * K, :],
                         preferred_element_type=jnp.float32)
            z += jnp.dot(s2[:, cs], wr[2 * K:3 * K, :],
                         preferred_element_type=jnp.float32)
            z += br[0:1, :]
            if act == "relu":
                z = jnp.maximum(z, 0.0)
            elif act == "softmax":
                m = jnp.max(z, axis=1, keepdims=True)
                e = jnp.exp(z - m)
                z = e / jnp.sum(e, axis=1, keepdims=True)
            o[:, pl.ds(b * N, N)] = z

    return pl.pallas_call(
        body,
        grid=(M // Mb,),
        in_specs=[
            pl.BlockSpec((Mb, 2 * K), lambda i: (i, 0)),
            pl.BlockSpec((Mb, 2 * K), lambda i: (i, 0)),
            pl.BlockSpec((Mb, 2 * K), lambda i: (i, 0)),
            pl.BlockSpec((3 * K, N), lambda i: (0, 0)),
            pl.BlockSpec((8, N), lambda i: (0, 0)),
        ],
        out_specs=pl.BlockSpec((Mb, 2 * N), lambda i: (i, 0)),
        out_shape=jax.ShapeDtypeStruct((M, 2 * N), jnp.float32),
    )


def _spmm(x2d, cols_p, rows_p, vals_p, meta, V, C2, nnzp):
    return _make_spmm(V, C2, nnzp)(x2d, cols_p, rows_p, vals_p, meta)


def _edge_prep(rows, cols, vals, V):
    nnz = rows.shape[0]
    nnzp = nnz + _PAD
    Vw = V // (2 * _NW)
    rows_p = jnp.concatenate([rows, jnp.full((_PAD,), V, jnp.int32)])
    cols_p = jnp.concatenate([cols, jnp.zeros((_PAD,), jnp.int32)])
    vals_p = jnp.concatenate([vals, jnp.zeros((_PAD,), jnp.float32)])
    bounds = jnp.searchsorted(
        rows, jnp.arange(0, V + 1, Vw, dtype=jnp.int32)).astype(jnp.int32)
    metas = []
    for h in range(2):
        b0 = bounds[h * _NW:(h + 1) * _NW]
        b1 = bounds[h * _NW + 1:(h + 1) * _NW + 1]
        metas += [(b0 // 8) * 8, b0, b1]
    meta = jnp.concatenate(metas)
    return cols_p, rows_p, vals_p, meta, nnzp


def _fold_weights(W, C, bn):
    wa, wb, wc = W[0:C], W[C:2 * C], W[2 * C:3 * C]
    wcat = jnp.concatenate([wa - wc, wb, 2.0 * wc], axis=0)
    n = W.shape[1]
    if bn is None:
        bias = jnp.zeros((n,), jnp.float32)
    else:
        gamma, beta, mean, var = bn
        scale = gamma * jax.lax.rsqrt(var + 1e-5)
        wcat = wcat * scale[None, :]
        bias = beta - mean * scale
    return wcat, jnp.broadcast_to(bias[None, :], (8, n))


def _mm_block(M):
    if M % 1024 == 0:
        return 1024
    return M


def _cheb_conv(x2d, rows, cols, vals, W, bn, act, V):
    C = x2d.shape[-1] // 2
    cols_p, rows_p, vals_p, meta, nnzp = _edge_prep(rows, cols, vals, V)
    s1 = _spmm(x2d, cols_p, rows_p, vals_p, meta, V, 2 * C, nnzp)
    s2 = _spmm(s1, cols_p, rows_p, vals_p, meta, V, 2 * C, nnzp)
    wcat, bias8 = _fold_weights(W, C, bn)
    n = W.shape[1]
    return _make_mm3(V, C, n, act, _mm_block(V))(x2d, s1, s2, wcat, bias8)


def kernel(x_enc0, x_enc1, x_enc2, x_enc3, x_enc4,
           lap1_rows, lap1_cols, lap1_vals,
           lap2_rows, lap2_cols, lap2_vals,
           lap3_rows, lap3_cols, lap3_vals,
           lap4_rows, lap4_cols, lap4_vals,
           lap5_rows, lap5_cols, lap5_vals,
           W1a, W1b, W2a, W2b, W3a, W3b, W4a, W4b, W5a, W5b,
           bn1a_gamma, bn1a_beta, bn1a_mean, bn1a_var,
           bn1b_gamma, bn1b_beta, bn1b_mean, bn1b_var,
           bn2a_gamma, bn2a_beta, bn2a_mean, bn2a_var,
           bn2b_gamma, bn2b_beta, bn2b_mean, bn2b_var,
           bn3a_gamma, bn3a_beta, bn3a_mean, bn3a_var,
           bn3b_gamma, bn3b_beta, bn3b_mean, bn3b_var,
           bn4a_gamma, bn4a_beta, bn4a_mean, bn4a_var,
           bn4b_gamma, bn4b_beta, bn4b_mean, bn4b_var,
           bn5a_gamma, bn5a_beta, bn5a_mean, bn5a_var):
    B = 2
    VS = [48, 192, 768, 3072, 12288, 49152]
    laps = {
        1: (lap1_rows, lap1_cols, lap1_vals),
        2: (lap2_rows, lap2_cols, lap2_vals),
        3: (lap3_rows, lap3_cols, lap3_vals),
        4: (lap4_rows, lap4_cols, lap4_vals),
        5: (lap5_rows, lap5_cols, lap5_vals),
    }
    encs = {1: x_enc1, 2: x_enc2, 3: x_enc3, 4: x_enc4}
    Ws = {"W1a": W1a, "W1b": W1b, "W2a": W2a, "W2b": W2b, "W3a": W3a,
          "W3b": W3b, "W4a": W4a, "W4b": W4b, "W5a": W5a, "W5b": W5b}
    bns = {
        "bn1a": (bn1a_gamma, bn1a_beta, bn1a_mean, bn1a_var),
        "bn1b": (bn1b_gamma, bn1b_beta, bn1b_mean, bn1b_var),
        "bn2a": (bn2a_gamma, bn2a_beta, bn2a_mean, bn2a_var),
        "bn2b": (bn2b_gamma, bn2b_beta, bn2b_mean, bn2b_var),
        "bn3a": (bn3a_gamma, bn3a_beta, bn3a_mean, bn3a_var),
        "bn3b": (bn3b_gamma, bn3b_beta, bn3b_mean, bn3b_var),
        "bn4a": (bn4a_gamma, bn4a_beta, bn4a_mean, bn4a_var),
        "bn4b": (bn4b_gamma, bn4b_beta, bn4b_mean, bn4b_var),
        "bn5a": (bn5a_gamma, bn5a_beta, bn5a_mean, bn5a_var),
    }

    def fold(a):
        return jnp.transpose(a, (1, 0, 2)).reshape(a.shape[1], -1)

    x = fold(x_enc0)
    for lvl in range(1, 5):
        rows, cols, vals = laps[lvl]
        V = VS[lvl]
        x = jnp.repeat(x, 4, axis=0)
        x = _cheb_conv(x, rows, cols, vals, Ws["W%da" % lvl],
                       bns["bn%da" % lvl], "relu", V)
        Ca = x.shape[1] // 2
        enc = fold(encs[lvl])
        Ce = enc.shape[1] // 2
        x = jnp.concatenate(
            [x[:, :Ca], enc[:, :Ce], x[:, Ca:], enc[:, Ce:]], axis=1)
        x = _cheb_conv(x, rows, cols, vals, Ws["W%db" % lvl],
                       bns["bn%db" % lvl], "relu", V)
    rows, cols, vals = laps[5]
    V = VS[5]
    x = jnp.repeat(x, 4, axis=0)
    x = _cheb_conv(x, rows, cols, vals, W5a, bns["bn5a"], "relu", V)
    x = _cheb_conv(x, rows, cols, vals, W5b, None, "softmax", V)
    return jnp.stack([x[:, :3], x[:, 3:]], axis=0)


# speedup vs baseline: 1.0064x; 1.0064x over previous
import functools

import jax
import jax.numpy as jnp
from jax import lax
from jax.experimental import pallas as pl
from jax.experimental.pallas import tpu as pltpu
from jax.experimental.pallas import tpu_sc as plsc

_NC = 2
_NS = 16
_NW = _NC * _NS
_PAD = 1024


_SPMEM_WORDS = 2097151


def _buf_sizes(acc_rows, Vw, C, nnz):
    ZR = 1
    for d in range(1, Vw + 1):
        if Vw % d == 0 and d * C <= 4096:
            ZR = d
    acc = acc_rows * C
    tile_words = (_SPMEM_WORDS - acc - 65536) // 16
    avail = tile_words - ZR * C - 512
    G = min(128, (avail // (2 * C + 31)) // 16 * 16)
    if G < 16:
        raise ValueError("no TileSpmem budget for acc_rows=%d C=%d" % (acc_rows, C))
    SCN = 2 if nnz <= 2 * _NW * G else 4
    return G, ZR, SCN


@functools.lru_cache(maxsize=None)
def _make_spmm(V, C, nnzp):
    V4 = V // 4
    VH = V // 2
    Vw = V // (2 * _NW)
    G, ZR, SCN = _buf_sizes(V4, Vw, C, nnzp - _PAD)
    NSUB = G // 16
    q = Vw // ZR
    EB = SCN * G
    NCG = C // 16

    mesh = plsc.VectorSubcoreMesh(core_axis_name="c", subcore_axis_name="s")

    def body(x_hbm, cols_hbm, rows_hbm, vals_hbm, meta_hbm, y_hbm,
             acc, g0, g1, ecols, erows, evals, cb0, cb1, vs, lr0, lr1,
             zbuf, metab, sem0, sem1, ssc0, ssc1):
        cid = lax.axis_index("c")
        sid = lax.axis_index("s")
        w = cid * _NS + sid
        lane = lax.iota(jnp.int32, 16)

        pltpu.sync_copy(meta_hbm, metab)

        def ext(base):
            seg0 = metab[pl.ds(base, 16)]
            seg1 = metab[pl.ds(base + 16, 16)]
            seg = jnp.where(jnp.full((16,), cid, jnp.int32) == 0, seg0, seg1)
            return jnp.sum(jnp.where(lane == sid, seg, 0))

        z16 = jnp.zeros((16,), jnp.float32)

        def zrow(r, _):
            def zcol(j, _):
                zbuf[r, pl.ds(j * 16, 16)] = z16
                return 0
            return lax.fori_loop(0, NCG, zcol, 0)
        lax.fori_loop(0, ZR, zrow, 0)

        own_lo = sid * Vw
        gbuf = [g0, g1]
        cbuf = [cb0, cb1]
        lrb = [lr0, lr1]
        sems = [sem0, sem1]
        sscs = [ssc0, ssc1]

        for h in range(2):
            a0 = ext(h * 96)
            b0 = ext(h * 96 + 32)
            b1 = ext(h * 96 + 64)
            nch = (b1 - a0 + (G - 1)) // G
            nsup = (nch + (SCN - 1)) // SCN
            rbase = h * VH + cid * V4

            def load_block(sup):
                off = pl.multiple_of(a0 + sup * EB, 8)
                pltpu.sync_copy(cols_hbm.at[pl.ds(off, EB)], ecols)
                pltpu.sync_copy(rows_hbm.at[pl.ds(off, EB)], erows)
                pltpu.sync_copy(vals_hbm.at[pl.ds(off, EB)], evals)

            def issue(m, p):
                for s in range(NSUB):
                    cbuf[p][pl.ds(s * 16, 16)] = ecols[pl.ds(m * G + s * 16, 16)]
                pltpu.async_copy(x_hbm.at[cbuf[p]], gbuf[p], sems[p])

            def zcp(i, _):
                pltpu.sync_copy(zbuf, acc.at[pl.ds(own_lo + i * ZR, ZR)])
                return 0
            lax.fori_loop(0, q, zcp, 0)

            @pl.when(nsup > 0)
            def _():
                load_block(0)
                issue(0, 0)

            def sup_body(sup, _):
                for m in range(SCN):
                    p = m % 2
                    pltpu.make_async_copy(
                        x_hbm.at[cbuf[p]], gbuf[p], sems[p]).wait()

                    base = a0 + (sup * SCN + m) * G
                    for s in range(NSUB):
                        ei = base + s * 16 + lane
                        valid = (ei >= b0) & (ei < b1)
                        vv = jnp.where(
                            valid, evals[pl.ds(m * G + s * 16, 16)], 0.0)
                        vs[pl.ds(s * 16, 16)] = vv
                        rr = erows[pl.ds(m * G + s * 16, 16)] - rbase
                        rr = jnp.clip(rr, own_lo, own_lo + (Vw - 1))
                        lrb[p][pl.ds(s * 16, 16)] = rr

                    def wait_sc(pp):
                        pltpu.make_async_copy(
                            gbuf[pp], acc.at[lrb[pp]], sscs[pp]).wait()

                    if m == 0:
                        @pl.when(sup > 0)
                        def _():
                            wait_sc(1 - p)
                        issue(m + 1, 1 - p)
                    elif m < SCN - 1:
                        wait_sc(1 - p)
                        issue(m + 1, 1 - p)
                    else:
                        @pl.when(sup + 1 < nsup)
                        def _():
                            wait_sc(1 - p)
                            load_block(sup + 1)
                            issue(0, 1 - p)

                    gp = gbuf[p]

                    @plsc.parallel_loop(0, G, unroll=4)
                    def _(j):
                        vsj = plsc.load_gather(
                            vs, [jnp.full((16,), j, jnp.int32)])
                        for cg in range(NCG):
                            gp[j, pl.ds(cg * 16, 16)] = (
                                gp[j, pl.ds(cg * 16, 16)] * vsj)

                    pltpu.async_copy(
                        gbuf[p], acc.at[lrb[p]], sscs[p], add=True)
                return 0
            lax.fori_loop(0, nsup, sup_body, 0)

            @pl.when(nsup > 0)
            def _():
                pltpu.make_async_copy(gbuf[0], acc.at[lrb[0]], sscs[0]).wait()
                pltpu.make_async_copy(gbuf[1], acc.at[lrb[1]], sscs[1]).wait()

            pltpu.sync_copy(acc.at[pl.ds(own_lo, Vw)],
                            y_hbm.at[pl.ds(h * VH + w * Vw, Vw)])

    return pl.kernel(
        body,
        out_type=jax.ShapeDtypeStruct((V, C), jnp.float32),
        mesh=mesh,
        scratch_types=[
            pltpu.VMEM_SHARED((V4, C), jnp.float32),
            pltpu.VMEM((G, C), jnp.float32),
            pltpu.VMEM((G, C), jnp.float32),
            pltpu.VMEM((EB,), jnp.int32),
            pltpu.VMEM((EB,), jnp.int32),
            pltpu.VMEM((EB,), jnp.float32),
            pltpu.VMEM((G,), jnp.int32),
            pltpu.VMEM((G,), jnp.int32),
            pltpu.VMEM((G,), jnp.float32),
            pltpu.VMEM((G,), jnp.int32),
            pltpu.VMEM((G,), jnp.int32),
            pltpu.VMEM((ZR, C), jnp.float32),
            pltpu.VMEM((192,), jnp.int32),
            pltpu.SemaphoreType.DMA,
            pltpu.SemaphoreType.DMA,
            pltpu.SemaphoreType.DMA,
            pltpu.SemaphoreType.DMA,
        ],
        compiler_params=pltpu.CompilerParams(use_tc_tiling_on_sc=False, needs_layout_passes=False),
    )


@functools.lru_cache(maxsize=None)
def _make_mm3(M, K, N, act, Mb):
    def body(x0, s1, s2, wr, br, o):
        for b in range(2):
            cs = pl.ds(b * K, K)
            z = jnp.dot(x0[:, cs], wr[0:K, :],
                        preferred_element_type=jnp.float32)
            z += jnp.dot(s1[:, cs], wr[K:2 * K, :],
                         preferred_element_type=jnp.float32)
            z += jnp.dot(s2[:, cs], wr[2 * K:3 * K, :],
                         preferred_element_type=jnp.float32)
            z += br[0:1, :]
            if act == "relu":
                z = jnp.maximum(z, 0.0)
            elif act == "softmax":
                m = jnp.max(z, axis=1, keepdims=True)
                e = jnp.exp(z - m)
                z = e / jnp.sum(e, axis=1, keepdims=True)
            o[:, pl.ds(b * N, N)] = z

    return pl.pallas_call(
        body,
        grid=(M // Mb,),
        in_specs=[
            pl.BlockSpec((Mb, 2 * K), lambda i: (i, 0)),
            pl.BlockSpec((Mb, 2 * K), lambda i: (i, 0)),
            pl.BlockSpec((Mb, 2 * K), lambda i: (i, 0)),
            pl.BlockSpec((3 * K, N), lambda i: (0, 0)),
            pl.BlockSpec((8, N), lambda i: (0, 0)),
        ],
        out_specs=pl.BlockSpec((Mb, 2 * N), lambda i: (i, 0)),
        out_shape=jax.ShapeDtypeStruct((M, 2 * N), jnp.float32),
    )


def _spmm(x2d, cols_p, rows_p, vals_p, meta, V, C2, nnzp):
    return _make_spmm(V, C2, nnzp)(x2d, cols_p, rows_p, vals_p, meta)


def _edge_prep(rows, cols, vals, V):
    nnz = rows.shape[0]
    nnzp = nnz + _PAD
    Vw = V // (2 * _NW)
    rows_p = jnp.concatenate([rows, jnp.full((_PAD,), V, jnp.int32)])
    cols_p = jnp.concatenate([cols, jnp.zeros((_PAD,), jnp.int32)])
    vals_p = jnp.concatenate([vals, jnp.zeros((_PAD,), jnp.float32)])
    bounds = jnp.searchsorted(
        rows, jnp.arange(0, V + 1, Vw, dtype=jnp.int32)).astype(jnp.int32)
    metas = []
    for h in range(2):
        b0 = bounds[h * _NW:(h + 1) * _NW]
        b1 = bounds[h * _NW + 1:(h + 1) * _NW + 1]
        metas += [(b0 // 8) * 8, b0, b1]
    meta = jnp.concatenate(metas)
    return cols_p, rows_p, vals_p, meta, nnzp


def _fold_weights(W, C, bn):
    wa, wb, wc = W[0:C], W[C:2 * C], W[2 * C:3 * C]
    wcat = jnp.concatenate([wa - wc, wb, 2.0 * wc], axis=0)
    n = W.shape[1]
    if bn is None:
        bias = jnp.zeros((n,), jnp.float32)
    else:
        gamma, beta, mean, var = bn
        scale = gamma * jax.lax.rsqrt(var + 1e-5)
        wcat = wcat * scale[None, :]
        bias = beta - mean * scale
    return wcat, jnp.broadcast_to(bias[None, :], (8, n))


def _mm_block(M):
    if M % 1024 == 0:
        return 1024
    return M


def _cheb_conv(x2d, rows, cols, vals, W, bn, act, V):
    C = x2d.shape[-1] // 2
    cols_p, rows_p, vals_p, meta, nnzp = _edge_prep(rows, cols, vals, V)
    s1 = _spmm(x2d, cols_p, rows_p, vals_p, meta, V, 2 * C, nnzp)
    s2 = _spmm(s1, cols_p, rows_p, vals_p, meta, V, 2 * C, nnzp)
    wcat, bias8 = _fold_weights(W, C, bn)
    n = W.shape[1]
    return _make_mm3(V, C, n, act, _mm_block(V))(x2d, s1, s2, wcat, bias8)


def kernel(x_enc0, x_enc1, x_enc2, x_enc3, x_enc4,
           lap1_rows, lap1_cols, lap1_vals,
           lap2_rows, lap2_cols, lap2_vals,
           lap3_rows, lap3_cols, lap3_vals,
           lap4_rows, lap4_cols, lap4_vals,
           lap5_rows, lap5_cols, lap5_vals,
           W1a, W1b, W2a, W2b, W3a, W3b, W4a, W4b, W5a, W5b,
           bn1a_gamma, bn1a_beta, bn1a_mean, bn1a_var,
           bn1b_gamma, bn1b_beta, bn1b_mean, bn1b_var,
           bn2a_gamma, bn2a_beta, bn2a_mean, bn2a_var,
           bn2b_gamma, bn2b_beta, bn2b_mean, bn2b_var,
           bn3a_gamma, bn3a_beta, bn3a_mean, bn3a_var,
           bn3b_gamma, bn3b_beta, bn3b_mean, bn3b_var,
           bn4a_gamma, bn4a_beta, bn4a_mean, bn4a_var,
           bn4b_gamma, bn4b_beta, bn4b_mean, bn4b_var,
           bn5a_gamma, bn5a_beta, bn5a_mean, bn5a_var):
    B = 2
    VS = [48, 192, 768, 3072, 12288, 49152]
    laps = {
        1: (lap1_rows, lap1_cols, lap1_vals),
        2: (lap2_rows, lap2_cols, lap2_vals),
        3: (lap3_rows, lap3_cols, lap3_vals),
        4: (lap4_rows, lap4_cols, lap4_vals),
        5: (lap5_rows, lap5_cols, lap5_vals),
    }
    encs = {1: x_enc1, 2: x_enc2, 3: x_enc3, 4: x_enc4}
    Ws = {"W1a": W1a, "W1b": W1b, "W2a": W2a, "W2b": W2b, "W3a": W3a,
          "W3b": W3b, "W4a": W4a, "W4b": W4b, "W5a": W5a, "W5b": W5b}
    bns = {
        "bn1a": (bn1a_gamma, bn1a_beta, bn1a_mean, bn1a_var),
        "bn1b": (bn1b_gamma, bn1b_beta, bn1b_mean, bn1b_var),
        "bn2a": (bn2a_gamma, bn2a_beta, bn2a_mean, bn2a_var),
        "bn2b": (bn2b_gamma, bn2b_beta, bn2b_mean, bn2b_var),
        "bn3a": (bn3a_gamma, bn3a_beta, bn3a_mean, bn3a_var),
        "bn3b": (bn3b_gamma, bn3b_beta, bn3b_mean, bn3b_var),
        "bn4a": (bn4a_gamma, bn4a_beta, bn4a_mean, bn4a_var),
        "bn4b": (bn4b_gamma, bn4b_beta, bn4b_mean, bn4b_var),
        "bn5a": (bn5a_gamma, bn5a_beta, bn5a_mean, bn5a_var),
    }

    def fold(a):
        return jnp.transpose(a, (1, 0, 2)).reshape(a.shape[1], -1)

    x = fold(x_enc0)
    for lvl in range(1, 5):
        rows, cols, vals = laps[lvl]
        V = VS[lvl]
        x = jnp.repeat(x, 4, axis=0)
        x = _cheb_conv(x, rows, cols, vals, Ws["W%da" % lvl],
                       bns["bn%da" % lvl], "relu", V)
        Ca = x.shape[1] // 2
        enc = fold(encs[lvl])
        Ce = enc.shape[1] // 2
        x = jnp.concatenate(
            [x[:, :Ca], enc[:, :Ce], x[:, Ca:], enc[:, Ce:]], axis=1)
        x = _cheb_conv(x, rows, cols, vals, Ws["W%db" % lvl],
                       bns["bn%db" % lvl], "relu", V)
    rows, cols, vals = laps[5]
    V = VS[5]
    x = jnp.repeat(x, 4, axis=0)
    x = _cheb_conv(x, rows, cols, vals, W5a, bns["bn5a"], "relu", V)
    x = _cheb_conv(x, rows, cols, vals, W5b, None, "softmax", V)
    return jnp.stack([x[:, :3], x[:, 3:]], axis=0)
